# baseline (speedup 1.0000x reference)
import jax
import jax.numpy as jnp
from jax import lax
from jax.experimental import pallas as pl
from jax.experimental.pallas import tpu as pltpu
from jax.experimental.pallas import tpu_sc as plsc

_BATCH = 16384
_K = 64
_CHUNK = 64
_NC = 2
_NS = 16
_NW = _NC * _NS
_BPW = _BATCH // _NW
_NCHUNK = _BPW // _CHUNK


def _pitf_body(*refs):
    (uid_hbm, iid_hbm, pid_hbm, nid_hbm,
     uV, iV, tuV, tiV, out_hbm) = refs[:9]
    idx_raw = refs[9:13]
    idx_pair = refs[13:17]
    colb = refs[17:21]
    bufs = refs[21:33]
    out_v = refs[33]
    sems = refs[34:36]

    wid = lax.axis_index("s") * _NC + lax.axis_index("c")
    base = wid * _BPW

    for t, src in enumerate((uid_hbm, iid_hbm, pid_hbm, nid_hbm)):
        pltpu.sync_copy(src.at[pl.ds(base, _BPW)], idx_raw[t])

    for t in range(4):
        raw, pair, cb = idx_raw[t], idx_pair[t], colb[t]

        @plsc.parallel_loop(0, _BPW // 16)
        def _split(i):
            v = raw[pl.ds(i * 16, 16)]
            pair[pl.ds(i * 16, 16)] = v >> 1
            cb[pl.ds(i * 16, 16)] = (v & 1) * _K

    def start(c):
        p = c % 2
        b = bufs[6 * p:6 * p + 6]
        sl = pl.ds(c * _CHUNK, _CHUNK)
        pairs = ((uV, idx_pair[0]), (iV, idx_pair[1]), (tuV, idx_pair[2]),
                 (tiV, idx_pair[2]), (tuV, idx_pair[3]), (tiV, idx_pair[3]))
        return [pltpu.async_copy(tab.at[ix.at[sl]], b[t], sems[p])
                for t, (tab, ix) in enumerate(pairs)]

    def compute(c):
        p = c % 2
        b_u, b_i, b_tup, b_tip, b_tun, b_tin = bufs[6 * p:6 * p + 6]

        def g_body(g, carry):
            idxr = g * 16 + lax.iota(jnp.int32, 16)
            row0 = c * _CHUNK + g * 16
            cb_u = colb[0][pl.ds(row0, 16)]
            cb_i = colb[1][pl.ds(row0, 16)]
            cb_p = colb[2][pl.ds(row0, 16)]
            cb_n = colb[3][pl.ds(row0, 16)]

            @plsc.parallel_loop(0, _K, unroll=4,
                                carry=jnp.zeros((16,), jnp.float32))
            def j_body(j, acc):
                u = plsc.load_gather(b_u, [idxr, cb_u + j])
                it = plsc.load_gather(b_i, [idxr, cb_i + j])
                tup = plsc.load_gather(b_tup, [idxr, cb_p + j])
                tip = plsc.load_gather(b_tip, [idxr, cb_p + j])
                tun = plsc.load_gather(b_tun, [idxr, cb_n + j])
                tin = plsc.load_gather(b_tin, [idxr, cb_n + j])
                return acc + u * (tup - tun) + it * (tip - tin)

            out_v[pl.ds(row0, 16)] = j_body
            return carry

        lax.fori_loop(0, _CHUNK // 16, g_body, 0)

    descs = [None, None]
    descs[0] = start(0)
    for c in range(_NCHUNK):
        if c + 1 < _NCHUNK:
            descs[(c + 1) % 2] = start(c + 1)
        for d in descs[c % 2]:
            d.wait()
        compute(c)

    pltpu.sync_copy(out_v, out_hbm.at[pl.ds(base, _BPW)])


_scratch = ([pltpu.VMEM((_BPW,), jnp.int32)] * 12
            + [pltpu.VMEM((_CHUNK, 2 * _K), jnp.float32)] * 12
            + [pltpu.VMEM((_BPW,), jnp.float32)]
            + [pltpu.SemaphoreType.DMA] * 2)

_pitf = pl.kernel(
    _pitf_body,
    out_type=jax.ShapeDtypeStruct((_BATCH,), jnp.float32),
    mesh=plsc.VectorSubcoreMesh(core_axis_name="c", subcore_axis_name="s"),
    scratch_types=_scratch,
    compiler_params=pltpu.CompilerParams(needs_layout_passes=False,
                                         use_tc_tiling_on_sc=True),
)


def kernel(x, userVecs, itemVecs, tagUserVecs, tagItemVecs):
    if x.ndim == 1:
        x = x.reshape(1, x.shape[0])
    uid = x[:, 0]
    iid = x[:, 1]
    pid = x[:, 2]
    nid = x[:, 3]
    return _pitf(uid, iid, pid, nid,
                 userVecs.reshape(-1, 2 * _K), itemVecs.reshape(-1, 2 * _K),
                 tagUserVecs.reshape(-1, 2 * _K),
                 tagItemVecs.reshape(-1, 2 * _K))


# speedup vs baseline: 1.8402x; 1.8402x over previous
import jax
import jax.numpy as jnp
from jax import lax
from jax.experimental import pallas as pl
from jax.experimental.pallas import tpu as pltpu
from jax.experimental.pallas import tpu_sc as plsc

_BATCH = 16384
_K = 64
_CHUNK = 64
_NC = 2
_NS = 16
_NW = _NC * _NS
_BPW = _BATCH // _NW
_NCHUNK = _BPW // _CHUNK


def _pitf_body(*refs):
    (uid_hbm, iid_hbm, pid_hbm, nid_hbm,
     uV, iV, tuV, tiV, out_hbm) = refs[:9]
    idx = refs[9:13]
    bufs = refs[13:25]
    out_v = refs[25]
    sems = refs[26:28]

    wid = lax.axis_index("s") * _NC + lax.axis_index("c")
    base = wid * _BPW

    for t, src in enumerate((uid_hbm, iid_hbm, pid_hbm, nid_hbm)):
        pltpu.sync_copy(src.at[pl.ds(base, _BPW)], idx[t])

    idx_u, idx_i, idx_p, idx_n = idx

    def start(c):
        p = c % 2
        b_u, b_i, b_tup, b_tip, b_tun, b_tin = bufs[6 * p:6 * p + 6]
        sem = sems[p]

        def g_body(g, carry):
            sl = pl.ds(c * _CHUNK + g * 16, 16)
            vu = idx_u[sl]
            vi = idx_i[sl]
            vp = idx_p[sl]
            vn = idx_n[sl]
            for k in range(16):
                r = g * 16 + k
                su = jnp.squeeze(lax.slice(vu, (k,), (k + 1,)))
                si = jnp.squeeze(lax.slice(vi, (k,), (k + 1,)))
                sp = jnp.squeeze(lax.slice(vp, (k,), (k + 1,)))
                sn = jnp.squeeze(lax.slice(vn, (k,), (k + 1,)))
                pltpu.async_copy(uV.at[su], b_u.at[r], sem)
                pltpu.async_copy(iV.at[si], b_i.at[r], sem)
                pltpu.async_copy(tuV.at[sp], b_tup.at[r], sem)
                pltpu.async_copy(tiV.at[sp], b_tip.at[r], sem)
                pltpu.async_copy(tuV.at[sn], b_tun.at[r], sem)
                pltpu.async_copy(tiV.at[sn], b_tin.at[r], sem)
            return carry

        lax.fori_loop(0, _CHUNK // 16, g_body, 0)

    def drain(c):
        p = c % 2
        for b in bufs[6 * p:6 * p + 6]:
            pltpu.make_async_copy(uV.at[pl.ds(0, _CHUNK)], b, sems[p]).wait()

    def compute(c):
        p = c % 2
        b_u, b_i, b_tup, b_tip, b_tun, b_tin = bufs[6 * p:6 * p + 6]

        last = lax.iota(jnp.int32, 16) == 15

        @plsc.parallel_loop(0, _CHUNK, unroll=4)
        def r_body(r):
            acc = jnp.zeros((16,), jnp.float32)
            for s in range(4):
                sl = pl.ds(s * 16, 16)
                u = b_u[r, sl]
                it = b_i[r, sl]
                tup = b_tup[r, sl]
                tip = b_tip[r, sl]
                tun = b_tun[r, sl]
                tin = b_tin[r, sl]
                acc = acc + u * (tup - tun) + it * (tip - tin)
            csum = plsc.cumsum(acc)
            pos = jnp.full((16,), c * _CHUNK + r, jnp.int32)
            plsc.store_scatter(out_v, [pos], csum, mask=last)

    start(0)
    for c in range(_NCHUNK):
        if c + 1 < _NCHUNK:
            start(c + 1)
        drain(c)
        compute(c)

    pltpu.sync_copy(out_v, out_hbm.at[pl.ds(base, _BPW)])


_scratch = ([pltpu.VMEM((_BPW,), jnp.int32)] * 4
            + [pltpu.VMEM((_CHUNK, _K), jnp.float32)] * 12
            + [pltpu.VMEM((_BPW,), jnp.float32)]
            + [pltpu.SemaphoreType.DMA] * 2)

_pitf = pl.kernel(
    _pitf_body,
    out_type=jax.ShapeDtypeStruct((_BATCH,), jnp.float32),
    mesh=plsc.VectorSubcoreMesh(core_axis_name="c", subcore_axis_name="s"),
    scratch_types=_scratch,
    compiler_params=pltpu.CompilerParams(needs_layout_passes=False,
                                         use_tc_tiling_on_sc=True),
)


def kernel(x, userVecs, itemVecs, tagUserVecs, tagItemVecs):
    if x.ndim == 1:
        x = x.reshape(1, x.shape[0])
    uid = x[:, 0]
    iid = x[:, 1]
    pid = x[:, 2]
    nid = x[:, 3]
    return _pitf(uid, iid, pid, nid,
                 userVecs, itemVecs, tagUserVecs, tagItemVecs)


# speedup vs baseline: 1.9109x; 1.0384x over previous
import jax
import jax.numpy as jnp
from jax import lax
from jax.experimental import pallas as pl
from jax.experimental.pallas import tpu as pltpu
from jax.experimental.pallas import tpu_sc as plsc

_BATCH = 16384
_K = 64
_CHUNK = 64
_NC = 2
_NS = 16
_NW = _NC * _NS
_BPW = _BATCH // _NW
_NCHUNK = _BPW // _CHUNK


def _make_half(with_prev):
    def body(*refs):
        n_in = 6 if with_prev else 5
        (aid_hbm, pid_hbm, nid_hbm, AV, TV) = refs[:5]
        prev_hbm = refs[5] if with_prev else None
        out_hbm = refs[n_in]
        sc = list(refs[n_in + 1:])
        idx = sc[0:3]
        bufs = sc[3:9]
        out_v = sc[9]
        prev_v = sc[10]
        sems = sc[11:13]

        wid = lax.axis_index("s") * _NC + lax.axis_index("c")
        base = wid * _BPW

        for t, src in enumerate((aid_hbm, pid_hbm, nid_hbm)):
            pltpu.sync_copy(src.at[pl.ds(base, _BPW)], idx[t])
        if with_prev:
            pltpu.sync_copy(prev_hbm.at[pl.ds(base, _BPW)],
                            prev_v.at[pl.ds(0, _BPW)])

        idx_a, idx_p, idx_n = idx

        def start(c):
            p = c % 2
            b_a, b_tp, b_tn = bufs[3 * p:3 * p + 3]
            sem = sems[p]

            def g_body(g, carry):
                sl = pl.ds(c * _CHUNK + g * 16, 16)
                va = idx_a[sl]
                vp = idx_p[sl]
                vn = idx_n[sl]
                for k in range(16):
                    r = g * 16 + k
                    sa = jnp.squeeze(lax.slice(va, (k,), (k + 1,)))
                    sp = jnp.squeeze(lax.slice(vp, (k,), (k + 1,)))
                    sn = jnp.squeeze(lax.slice(vn, (k,), (k + 1,)))
                    pltpu.async_copy(AV.at[sa], b_a.at[r], sem)
                    pltpu.async_copy(TV.at[sp], b_tp.at[r], sem)
                    pltpu.async_copy(TV.at[sn], b_tn.at[r], sem)
                return carry

            lax.fori_loop(0, _CHUNK // 16, g_body, 0)

        def drain(c):
            p = c % 2
            for b in bufs[3 * p:3 * p + 3]:
                pltpu.make_async_copy(AV.at[pl.ds(0, _CHUNK)], b,
                                      sems[p]).wait()

        def compute(c):
            p = c % 2
            b_a, b_tp, b_tn = bufs[3 * p:3 * p + 3]

            lanes = lax.iota(jnp.int32, 16)
            last = lanes == 15
            first = lanes == 0
            zeros = jnp.zeros((16,), jnp.float32)

            @plsc.parallel_loop(0, _CHUNK, unroll=4)
            def r_body(r):
                acc = zeros
                for s in range(4):
                    sl = pl.ds(s * 16, 16)
                    a = b_a[r, sl]
                    tp = b_tp[r, sl]
                    tn = b_tn[r, sl]
                    acc = acc + a * (tp - tn)
                if with_prev:
                    pv = prev_v[pl.ds(c * _CHUNK + r, 16)]
                    acc = acc + jnp.where(first, pv, zeros)
                csum = plsc.cumsum(acc)
                pos = jnp.full((16,), c * _CHUNK + r, jnp.int32)
                plsc.store_scatter(out_v, [pos], csum, mask=last)

        start(0)
        for c in range(_NCHUNK):
            if c + 1 < _NCHUNK:
                start(c + 1)
            drain(c)
            compute(c)

        pltpu.sync_copy(out_v, out_hbm.at[pl.ds(base, _BPW)])

    scratch = ([pltpu.VMEM((_BPW,), jnp.int32)] * 3
               + [pltpu.VMEM((_CHUNK, _K), jnp.float32)] * 6
               + [pltpu.VMEM((_BPW,), jnp.float32)]
               + [pltpu.VMEM((_BPW + 16,), jnp.float32)]
               + [pltpu.SemaphoreType.DMA] * 2)

    return pl.kernel(
        body,
        out_type=jax.ShapeDtypeStruct((_BATCH,), jnp.float32),
        mesh=plsc.VectorSubcoreMesh(core_axis_name="c",
                                    subcore_axis_name="s"),
        scratch_types=scratch,
        compiler_params=pltpu.CompilerParams(needs_layout_passes=False,
                                             use_tc_tiling_on_sc=True),
    )


_half_a = _make_half(False)
_half_b = _make_half(True)


def kernel(x, userVecs, itemVecs, tagUserVecs, tagItemVecs):
    if x.ndim == 1:
        x = x.reshape(1, x.shape[0])
    uid = x[:, 0]
    iid = x[:, 1]
    pid = x[:, 2]
    nid = x[:, 3]
    part = _half_a(uid, pid, nid, userVecs, tagUserVecs)
    return _half_b(iid, pid, nid, itemVecs, tagItemVecs, part)


# speedup vs baseline: 1.9305x; 1.0103x over previous
import jax
import jax.numpy as jnp
from jax import lax
from jax.experimental import pallas as pl
from jax.experimental.pallas import tpu as pltpu
from jax.experimental.pallas import tpu_sc as plsc

_BATCH = 16384
_K = 64
_CHUNK = 128
_NC = 2
_NS = 16
_NW = _NC * _NS
_BPW = _BATCH // _NW
_NCHUNK = _BPW // _CHUNK


def _make_half(with_prev):
    def body(*refs):
        n_in = 6 if with_prev else 5
        (aid_hbm, pid_hbm, nid_hbm, AV, TV) = refs[:5]
        prev_hbm = refs[5] if with_prev else None
        out_hbm = refs[n_in]
        sc = list(refs[n_in + 1:])
        idx = sc[0:3]
        bufs = sc[3:9]
        out_v = sc[9]
        prev_v = sc[10]
        sems = sc[11:13]

        wid = lax.axis_index("s") * _NC + lax.axis_index("c")
        base = wid * _BPW

        for t, src in enumerate((aid_hbm, pid_hbm, nid_hbm)):
            pltpu.sync_copy(src.at[pl.ds(base, _BPW)], idx[t])
        if with_prev:
            pltpu.sync_copy(prev_hbm.at[pl.ds(base, _BPW)],
                            prev_v.at[pl.ds(0, _BPW)])

        idx_a, idx_p, idx_n = idx

        def start(c):
            p = c % 2
            b_a, b_tp, b_tn = bufs[3 * p:3 * p + 3]
            sem = sems[p]

            def g_body(g, carry):
                sl = pl.ds(c * _CHUNK + g * 16, 16)
                va = idx_a[sl]
                vp = idx_p[sl]
                vn = idx_n[sl]
                for k in range(16):
                    r = g * 16 + k
                    sa = jnp.squeeze(lax.slice(va, (k,), (k + 1,)))
                    sp = jnp.squeeze(lax.slice(vp, (k,), (k + 1,)))
                    sn = jnp.squeeze(lax.slice(vn, (k,), (k + 1,)))
                    pltpu.async_copy(AV.at[sa], b_a.at[r], sem)
                    pltpu.async_copy(TV.at[sp], b_tp.at[r], sem)
                    pltpu.async_copy(TV.at[sn], b_tn.at[r], sem)
                return carry

            lax.fori_loop(0, _CHUNK // 16, g_body, 0)

        def drain(c):
            p = c % 2
            for b in bufs[3 * p:3 * p + 3]:
                pltpu.make_async_copy(AV.at[pl.ds(0, _CHUNK)], b,
                                      sems[p]).wait()

        def compute(c):
            p = c % 2
            b_a, b_tp, b_tn = bufs[3 * p:3 * p + 3]

            lanes = lax.iota(jnp.int32, 16)
            last = lanes == 15
            first = lanes == 0
            zeros = jnp.zeros((16,), jnp.float32)

            @plsc.parallel_loop(0, _CHUNK, unroll=4)
            def r_body(r):
                acc = zeros
                for s in range(4):
                    sl = pl.ds(s * 16, 16)
                    a = b_a[r, sl]
                    tp = b_tp[r, sl]
                    tn = b_tn[r, sl]
                    acc = acc + a * (tp - tn)
                if with_prev:
                    pv = prev_v[pl.ds(c * _CHUNK + r, 16)]
                    acc = acc + jnp.where(first, pv, zeros)
                csum = plsc.cumsum(acc)
                pos = jnp.full((16,), c * _CHUNK + r, jnp.int32)
                plsc.store_scatter(out_v, [pos], csum, mask=last)

        start(0)
        for c in range(_NCHUNK):
            if c + 1 < _NCHUNK:
                start(c + 1)
            drain(c)
            compute(c)

        pltpu.sync_copy(out_v, out_hbm.at[pl.ds(base, _BPW)])

    scratch = ([pltpu.VMEM((_BPW,), jnp.int32)] * 3
               + [pltpu.VMEM((_CHUNK, _K), jnp.float32)] * 6
               + [pltpu.VMEM((_BPW,), jnp.float32)]
               + [pltpu.VMEM((_BPW + 16,), jnp.float32)]
               + [pltpu.SemaphoreType.DMA] * 2)

    return pl.kernel(
        body,
        out_type=jax.ShapeDtypeStruct((_BATCH,), jnp.float32),
        mesh=plsc.VectorSubcoreMesh(core_axis_name="c",
                                    subcore_axis_name="s"),
        scratch_types=scratch,
        compiler_params=pltpu.CompilerParams(needs_layout_passes=False,
                                             use_tc_tiling_on_sc=True),
    )


_half_a = _make_half(False)
_half_b = _make_half(True)


def kernel(x, userVecs, itemVecs, tagUserVecs, tagItemVecs):
    if x.ndim == 1:
        x = x.reshape(1, x.shape[0])
    uid = x[:, 0]
    iid = x[:, 1]
    pid = x[:, 2]
    nid = x[:, 3]
    part = _half_a(uid, pid, nid, userVecs, tagUserVecs)
    return _half_b(iid, pid, nid, itemVecs, tagItemVecs, part)
